# baseline (device time: 120123 ns/iter reference)
import functools

import jax
import jax.numpy as jnp
from jax import lax
from jax.experimental import pallas as pl
from jax.experimental.pallas import tpu as pltpu

B = 2
H = 256
W = 256
C = 128
N_GLOBAL = 512 * 512
EPS = 1e-5

T1 = 32
T = 64
R = H // T


def _stats_body(x_ref, sum_ref, sq_ref, xb_ref, colsend_ref):
    t = pl.program_id(0)
    py = lax.axis_index("y")
    blk = x_ref[...]
    xbv = blk.astype(jnp.bfloat16)
    xb_ref[...] = xbv
    colsend_ref[...] = jnp.where(
        py == 0, xbv[:, :, W - 1:W, :], xbv[:, :, 0:1, :])
    s = jnp.sum(blk, axis=(1, 2))
    q = jnp.sum(blk * blk, axis=(1, 2))

    @pl.when(t == 0)
    def _():
        sum_ref[...] = s
        sq_ref[...] = q

    @pl.when(t != 0)
    def _():
        sum_ref[...] = sum_ref[...] + s
        sq_ref[...] = sq_ref[...] + q


def _exchange_body(
    x_ref, colsend_ref, psum_ref, psq_ref,
    stats_ref, hrow_ref, hcol_ref, hcorn_ref,
    row_b3,
    row_bf, col_bf, corn_bf, stats_send, stats_recv,
    local_sems, send_sems, recv_sems,
):
    px = lax.axis_index("x")
    py = lax.axis_index("y")
    rs = (1 - px) * (H - 1)

    cp_row = pltpu.make_async_copy(
        x_ref.at[:, pl.ds(rs, 1)], row_b3, local_sems.at[0])
    cp_row.start()
    col_bf[...] = colsend_ref[:, :, 0]
    corn_bf[...] = colsend_ref[:, pl.ds(rs, 1), 0]
    stats_send[0] = psum_ref[...]
    stats_send[1] = psq_ref[...]
    cp_row.wait()
    row_bf[...] = row_b3[:, 0]

    xn = (1 - px, py)
    yn = (px, 1 - py)
    dn = (1 - px, 1 - py)
    nbrs = [xn, yn, dn]

    barrier = pltpu.get_barrier_semaphore()
    for d in nbrs:
        pl.semaphore_signal(
            barrier, inc=1, device_id=d, device_id_type=pl.DeviceIdType.MESH)
    pl.semaphore_wait(barrier, 3)

    rdmas = []
    rdmas.append(pltpu.make_async_remote_copy(
        row_bf, hrow_ref, send_sems.at[0], recv_sems.at[0],
        device_id=xn, device_id_type=pl.DeviceIdType.MESH))
    rdmas.append(pltpu.make_async_remote_copy(
        col_bf, hcol_ref, send_sems.at[1], recv_sems.at[1],
        device_id=yn, device_id_type=pl.DeviceIdType.MESH))
    rdmas.append(pltpu.make_async_remote_copy(
        corn_bf, hcorn_ref, send_sems.at[2], recv_sems.at[2],
        device_id=dn, device_id_type=pl.DeviceIdType.MESH))
    for i, d in enumerate(nbrs):
        rdmas.append(pltpu.make_async_remote_copy(
            stats_send, stats_recv.at[i], send_sems.at[3 + i],
            recv_sems.at[3 + i], device_id=d,
            device_id_type=pl.DeviceIdType.MESH))
    for r in rdmas:
        r.start()
    for r in rdmas:
        r.wait()

    tot = psum_ref[...] + stats_recv[0, 0] + stats_recv[1, 0] + stats_recv[2, 0]
    tsq = psq_ref[...] + stats_recv[0, 1] + stats_recv[1, 1] + stats_recv[2, 1]
    mean = tot / N_GLOBAL
    var = tsq / N_GLOBAL - mean * mean
    stats_ref[0] = mean[:, None, :]
    stats_ref[1] = lax.rsqrt(var + EPS)[:, None, :]

    @functools.partial(pl.run_scoped, sem2=pltpu.SemaphoreType.REGULAR)
    def _(sem2):
        for d in nbrs:
            pl.semaphore_signal(
                sem2, inc=1, device_id=d, device_id_type=pl.DeviceIdType.MESH)
        pl.semaphore_wait(sem2, 3)


def _compute_body(
    x_tile_ref, x_any_ref, stats_ref, hrow_ref, hcol_ref, hcorn_ref,
    k_ref, wp_ref, out_ref, top_buf, bot_buf, sems,
):
    b = pl.program_id(0)
    t = pl.program_id(1)
    px = lax.axis_index("x")
    py = lax.axis_index("y")
    r0 = t * T
    ta = jnp.maximum(r0 - 1, 0)
    tb = jnp.minimum(r0 + T, H - 1)

    cp_t = pltpu.make_async_copy(
        x_any_ref.at[b, pl.ds(ta, 1)], top_buf, sems.at[0])
    cp_t.start()
    cp_b = pltpu.make_async_copy(
        x_any_ref.at[b, pl.ds(tb, 1)], bot_buf, sems.at[1])
    cp_b.start()

    mean = stats_ref[0, b, 0].astype(jnp.bfloat16)
    rstd = stats_ref[1, b, 0].astype(jnp.bfloat16)

    def nrm(v):
        return (v.astype(jnp.bfloat16) - mean) * rstd

    def pick_row(mat, i):
        idx = lax.broadcasted_iota(jnp.int32, mat.shape, 0)
        return jnp.sum(jnp.where(idx == i, mat, jnp.zeros_like(mat)), axis=0)

    xb = x_tile_ref[0]
    h_mid = (xb - mean) * rstd

    hc_all = nrm(hcol_ref[b])
    hcolv = nrm(hcol_ref[b, pl.ds(r0, T)])
    edge_col = jnp.where(py == 0, h_mid[:, 0, :], h_mid[:, W - 1, :])
    left_col = jnp.where(py == 0, edge_col, hcolv)
    right_col = jnp.where(py == 0, hcolv, edge_col)
    mid = jnp.concatenate(
        [left_col[:, None, :], h_mid, right_col[:, None, :]], axis=1)

    cp_t.wait()
    cp_b.wait()

    is_th = jnp.logical_and(t == 0, px == 1)
    top_base = jnp.where(is_th, nrm(hrow_ref[b]), nrm(top_buf[0]))
    he_t = jnp.where(is_th, nrm(hcorn_ref[b, 0]), pick_row(hc_all, ta))
    ee_t = jnp.where(py == 0, top_base[0], top_base[W - 1])
    lt = jnp.where(py == 0, ee_t, he_t)
    rt = jnp.where(py == 0, he_t, ee_t)
    s_top = jnp.concatenate([lt[None], top_base, rt[None]], axis=0)

    is_bh = jnp.logical_and(t == R - 1, px == 0)
    bot_base = jnp.where(is_bh, nrm(hrow_ref[b]), nrm(bot_buf[0]))
    he_b = jnp.where(is_bh, nrm(hcorn_ref[b, 0]), pick_row(hc_all, tb))
    ee_b = jnp.where(py == 0, bot_base[0], bot_base[W - 1])
    lb = jnp.where(py == 0, ee_b, he_b)
    rb = jnp.where(py == 0, he_b, ee_b)
    s_bot = jnp.concatenate([lb[None], bot_base, rb[None]], axis=0)

    ext = jnp.concatenate([s_top[None], mid, s_bot[None]], axis=0)

    kb = k_ref[...].astype(jnp.bfloat16)
    g = [
        ext[0:T] * kb[0, dj] + ext[1:T + 1] * kb[1, dj] + ext[2:T + 2] * kb[2, dj]
        for dj in range(3)
    ]
    acc = g[0][:, 0:W] + g[1][:, 1:W + 1] + g[2][:, 2:W + 2]

    a = acc * (0.5 * jnp.tanh(0.5 * acc) + 0.5)
    ab = a.reshape(T * W, C)
    wb = wp_ref[...].astype(jnp.bfloat16)
    mm = lax.dot_general(
        ab, wb, (((1,), (0,)), ((), ())),
        preferred_element_type=jnp.float32)
    out_ref[0] = xb + mm.reshape(T, W, C).astype(jnp.bfloat16)


def kernel(x, k, Wp):
    psum, psq, xb, colsend = pl.pallas_call(
        _stats_body,
        grid=(H // T1,),
        in_specs=[pl.BlockSpec((B, T1, W, C), lambda t: (0, t, 0, 0))],
        out_specs=[
            pl.BlockSpec((B, C), lambda t: (0, 0)),
            pl.BlockSpec((B, C), lambda t: (0, 0)),
            pl.BlockSpec((B, T1, W, C), lambda t: (0, t, 0, 0)),
            pl.BlockSpec((B, T1, 1, C), lambda t: (0, t, 0, 0)),
        ],
        out_shape=[
            jax.ShapeDtypeStruct((B, C), jnp.float32),
            jax.ShapeDtypeStruct((B, C), jnp.float32),
            jax.ShapeDtypeStruct((B, H, W, C), jnp.bfloat16),
            jax.ShapeDtypeStruct((B, H, 1, C), jnp.bfloat16),
        ],
        compiler_params=pltpu.CompilerParams(
            dimension_semantics=("arbitrary",)),
    )(x)

    stats, hrow, hcol, hcorn = pl.pallas_call(
        _exchange_body,
        in_specs=[
            pl.BlockSpec(memory_space=pl.ANY),
            pl.BlockSpec(memory_space=pltpu.VMEM),
            pl.BlockSpec(memory_space=pltpu.VMEM),
            pl.BlockSpec(memory_space=pltpu.VMEM),
        ],
        out_specs=[pl.BlockSpec(memory_space=pltpu.VMEM)] * 4,
        out_shape=[
            jax.ShapeDtypeStruct((2, B, 1, C), jnp.float32),
            jax.ShapeDtypeStruct((B, H, C), jnp.bfloat16),
            jax.ShapeDtypeStruct((B, W, C), jnp.bfloat16),
            jax.ShapeDtypeStruct((B, 1, C), jnp.bfloat16),
        ],
        scratch_shapes=[
            pltpu.VMEM((B, 1, W, C), jnp.bfloat16),
            pltpu.VMEM((B, H, C), jnp.bfloat16),
            pltpu.VMEM((B, W, C), jnp.bfloat16),
            pltpu.VMEM((B, 1, C), jnp.bfloat16),
            pltpu.VMEM((2, B, C), jnp.float32),
            pltpu.VMEM((3, 2, B, C), jnp.float32),
            pltpu.SemaphoreType.DMA((3,)),
            pltpu.SemaphoreType.DMA((6,)),
            pltpu.SemaphoreType.DMA((6,)),
        ],
        compiler_params=pltpu.CompilerParams(collective_id=0),
    )(xb, colsend, psum, psq)

    out = pl.pallas_call(
        _compute_body,
        grid=(B, R),
        in_specs=[
            pl.BlockSpec((1, T, W, C), lambda b, t: (b, t, 0, 0)),
            pl.BlockSpec(memory_space=pl.ANY),
            pl.BlockSpec(memory_space=pltpu.VMEM),
            pl.BlockSpec(memory_space=pltpu.VMEM),
            pl.BlockSpec(memory_space=pltpu.VMEM),
            pl.BlockSpec(memory_space=pltpu.VMEM),
            pl.BlockSpec(memory_space=pltpu.VMEM),
            pl.BlockSpec(memory_space=pltpu.VMEM),
        ],
        out_specs=pl.BlockSpec((1, T, W, C), lambda b, t: (b, t, 0, 0)),
        out_shape=jax.ShapeDtypeStruct((B, H, W, C), jnp.bfloat16),
        scratch_shapes=[
            pltpu.VMEM((1, W, C), jnp.bfloat16),
            pltpu.VMEM((1, W, C), jnp.bfloat16),
            pltpu.SemaphoreType.DMA((2,)),
        ],
        compiler_params=pltpu.CompilerParams(
            dimension_semantics=("arbitrary", "arbitrary"),
            vmem_limit_bytes=56 * 1024 * 1024),
    )(xb, xb, stats, hrow, hcol, hcorn, k, Wp)
    return out


# device time: 115678 ns/iter; 1.0384x vs baseline; 1.0384x over previous
import functools

import jax
import jax.numpy as jnp
from jax import lax
from jax.experimental import pallas as pl
from jax.experimental.pallas import tpu as pltpu

B = 2
H = 256
W = 256
C = 128
N_GLOBAL = 512 * 512
EPS = 1e-5

T1 = 32
T = 64
R = H // T


def _stats_body(x_ref, sum_ref, sq_ref):
    t = pl.program_id(0)
    blk = x_ref[...]
    s = jnp.sum(blk, axis=(1, 2))
    q = jnp.sum(blk * blk, axis=(1, 2))

    @pl.when(t == 0)
    def _():
        sum_ref[...] = s
        sq_ref[...] = q

    @pl.when(t != 0)
    def _():
        sum_ref[...] = sum_ref[...] + s
        sq_ref[...] = sq_ref[...] + q


def _exchange_body(
    x_ref, psum_ref, psq_ref,
    stats_ref, hrow_ref, hcol_ref, hcorn_ref,
    row_f32, col_f32, corn_f32,
    row_bf, col_bf, corn_bf, stats_send, stats_recv,
    local_sems, send_sems, recv_sems,
):
    px = lax.axis_index("x")
    py = lax.axis_index("y")
    rs = (1 - px) * (H - 1)
    cs = (1 - py) * (W - 1)

    cp_row = pltpu.make_async_copy(
        x_ref.at[:, pl.ds(rs, 1)], row_f32, local_sems.at[0])
    cp_row.start()
    cp_col = pltpu.make_async_copy(
        x_ref.at[:, :, pl.ds(cs, 1)], col_f32, local_sems.at[1])
    cp_col.start()
    cp_corn = pltpu.make_async_copy(
        x_ref.at[:, pl.ds(rs, 1), pl.ds(cs, 1)], corn_f32, local_sems.at[2])
    cp_corn.start()
    cp_row.wait()
    cp_col.wait()
    cp_corn.wait()

    row_bf[...] = row_f32[:, 0].astype(jnp.bfloat16)
    col_bf[...] = col_f32[:, :, 0].astype(jnp.bfloat16)
    corn_bf[...] = corn_f32[:, 0].astype(jnp.bfloat16)
    stats_send[0] = psum_ref[...]
    stats_send[1] = psq_ref[...]

    xn = (1 - px, py)
    yn = (px, 1 - py)
    dn = (1 - px, 1 - py)
    nbrs = [xn, yn, dn]

    barrier = pltpu.get_barrier_semaphore()
    for d in nbrs:
        pl.semaphore_signal(
            barrier, inc=1, device_id=d, device_id_type=pl.DeviceIdType.MESH)
    pl.semaphore_wait(barrier, 3)

    rdmas = []
    rdmas.append(pltpu.make_async_remote_copy(
        row_bf, hrow_ref, send_sems.at[0], recv_sems.at[0],
        device_id=xn, device_id_type=pl.DeviceIdType.MESH))
    rdmas.append(pltpu.make_async_remote_copy(
        col_bf, hcol_ref, send_sems.at[1], recv_sems.at[1],
        device_id=yn, device_id_type=pl.DeviceIdType.MESH))
    rdmas.append(pltpu.make_async_remote_copy(
        corn_bf, hcorn_ref, send_sems.at[2], recv_sems.at[2],
        device_id=dn, device_id_type=pl.DeviceIdType.MESH))
    for i, d in enumerate(nbrs):
        rdmas.append(pltpu.make_async_remote_copy(
            stats_send, stats_recv.at[i], send_sems.at[3 + i],
            recv_sems.at[3 + i], device_id=d,
            device_id_type=pl.DeviceIdType.MESH))
    for r in rdmas:
        r.start()
    for r in rdmas:
        r.wait()

    tot = psum_ref[...] + stats_recv[0, 0] + stats_recv[1, 0] + stats_recv[2, 0]
    tsq = psq_ref[...] + stats_recv[0, 1] + stats_recv[1, 1] + stats_recv[2, 1]
    mean = tot / N_GLOBAL
    var = tsq / N_GLOBAL - mean * mean
    stats_ref[0] = mean[:, None, :]
    stats_ref[1] = lax.rsqrt(var + EPS)[:, None, :]

    @functools.partial(pl.run_scoped, sem2=pltpu.SemaphoreType.REGULAR)
    def _(sem2):
        for d in nbrs:
            pl.semaphore_signal(
                sem2, inc=1, device_id=d, device_id_type=pl.DeviceIdType.MESH)
        pl.semaphore_wait(sem2, 3)


def _compute_body(
    x_tile_ref, x_any_ref, stats_ref, hrow_ref, hcol_ref, hcorn_ref,
    k_ref, wp_ref, out_ref, top_buf, bot_buf, sems,
):
    b = pl.program_id(0)
    t = pl.program_id(1)
    px = lax.axis_index("x")
    py = lax.axis_index("y")
    r0 = t * T
    ta = jnp.maximum(r0 - 1, 0)
    tb = jnp.minimum(r0 + T, H - 1)

    cp_t = pltpu.make_async_copy(
        x_any_ref.at[b, pl.ds(ta, 1)], top_buf, sems.at[0])
    cp_t.start()
    cp_b = pltpu.make_async_copy(
        x_any_ref.at[b, pl.ds(tb, 1)], bot_buf, sems.at[1])
    cp_b.start()

    mean = stats_ref[0, b, 0].astype(jnp.bfloat16)
    rstd = stats_ref[1, b, 0].astype(jnp.bfloat16)

    def nrm(v):
        return (v.astype(jnp.bfloat16) - mean) * rstd

    def pick_row(mat, i):
        idx = lax.broadcasted_iota(jnp.int32, mat.shape, 0)
        return jnp.sum(jnp.where(idx == i, mat, jnp.zeros_like(mat)), axis=0)

    xb = x_tile_ref[0].astype(jnp.bfloat16)
    h_mid = (xb - mean) * rstd

    hc_all = nrm(hcol_ref[b])
    hcolv = nrm(hcol_ref[b, pl.ds(r0, T)])
    edge_col = jnp.where(py == 0, h_mid[:, 0, :], h_mid[:, W - 1, :])
    left_col = jnp.where(py == 0, edge_col, hcolv)
    right_col = jnp.where(py == 0, hcolv, edge_col)
    mid = jnp.concatenate(
        [left_col[:, None, :], h_mid, right_col[:, None, :]], axis=1)

    cp_t.wait()
    cp_b.wait()

    is_th = jnp.logical_and(t == 0, px == 1)
    top_base = jnp.where(is_th, nrm(hrow_ref[b]), nrm(top_buf[0]))
    he_t = jnp.where(is_th, nrm(hcorn_ref[b, 0]), pick_row(hc_all, ta))
    ee_t = jnp.where(py == 0, top_base[0], top_base[W - 1])
    lt = jnp.where(py == 0, ee_t, he_t)
    rt = jnp.where(py == 0, he_t, ee_t)
    s_top = jnp.concatenate([lt[None], top_base, rt[None]], axis=0)

    is_bh = jnp.logical_and(t == R - 1, px == 0)
    bot_base = jnp.where(is_bh, nrm(hrow_ref[b]), nrm(bot_buf[0]))
    he_b = jnp.where(is_bh, nrm(hcorn_ref[b, 0]), pick_row(hc_all, tb))
    ee_b = jnp.where(py == 0, bot_base[0], bot_base[W - 1])
    lb = jnp.where(py == 0, ee_b, he_b)
    rb = jnp.where(py == 0, he_b, ee_b)
    s_bot = jnp.concatenate([lb[None], bot_base, rb[None]], axis=0)

    ext = jnp.concatenate([s_top[None], mid, s_bot[None]], axis=0)

    kb = k_ref[...].astype(jnp.bfloat16)
    g = [
        ext[0:T] * kb[0, dj] + ext[1:T + 1] * kb[1, dj] + ext[2:T + 2] * kb[2, dj]
        for dj in range(3)
    ]
    acc = g[0][:, 0:W] + g[1][:, 1:W + 1] + g[2][:, 2:W + 2]

    a = acc * (0.5 * jnp.tanh(0.5 * acc) + 0.5)
    ab = a.reshape(T * W, C)
    wb = wp_ref[...].astype(jnp.bfloat16)
    mm = lax.dot_general(
        ab, wb, (((1,), (0,)), ((), ())),
        preferred_element_type=jnp.float32)
    out_ref[0] = xb + mm.reshape(T, W, C).astype(jnp.bfloat16)


def kernel(x, k, Wp):
    psum, psq = pl.pallas_call(
        _stats_body,
        grid=(H // T1,),
        in_specs=[pl.BlockSpec((B, T1, W, C), lambda t: (0, t, 0, 0))],
        out_specs=[
            pl.BlockSpec((B, C), lambda t: (0, 0)),
            pl.BlockSpec((B, C), lambda t: (0, 0)),
        ],
        out_shape=[
            jax.ShapeDtypeStruct((B, C), jnp.float32),
            jax.ShapeDtypeStruct((B, C), jnp.float32),
        ],
        compiler_params=pltpu.CompilerParams(
            dimension_semantics=("arbitrary",)),
    )(x)

    stats, hrow, hcol, hcorn = pl.pallas_call(
        _exchange_body,
        in_specs=[
            pl.BlockSpec(memory_space=pl.ANY),
            pl.BlockSpec(memory_space=pltpu.VMEM),
            pl.BlockSpec(memory_space=pltpu.VMEM),
        ],
        out_specs=[pl.BlockSpec(memory_space=pltpu.VMEM)] * 4,
        out_shape=[
            jax.ShapeDtypeStruct((2, B, 1, C), jnp.float32),
            jax.ShapeDtypeStruct((B, H, C), jnp.bfloat16),
            jax.ShapeDtypeStruct((B, W, C), jnp.bfloat16),
            jax.ShapeDtypeStruct((B, 1, C), jnp.bfloat16),
        ],
        scratch_shapes=[
            pltpu.VMEM((B, 1, W, C), jnp.float32),
            pltpu.VMEM((B, H, 1, C), jnp.float32),
            pltpu.VMEM((B, 1, 1, C), jnp.float32),
            pltpu.VMEM((B, H, C), jnp.bfloat16),
            pltpu.VMEM((B, W, C), jnp.bfloat16),
            pltpu.VMEM((B, 1, C), jnp.bfloat16),
            pltpu.VMEM((2, B, C), jnp.float32),
            pltpu.VMEM((3, 2, B, C), jnp.float32),
            pltpu.SemaphoreType.DMA((3,)),
            pltpu.SemaphoreType.DMA((6,)),
            pltpu.SemaphoreType.DMA((6,)),
        ],
        compiler_params=pltpu.CompilerParams(collective_id=0),
    )(x, psum, psq)

    out = pl.pallas_call(
        _compute_body,
        grid=(B, R),
        in_specs=[
            pl.BlockSpec((1, T, W, C), lambda b, t: (b, t, 0, 0)),
            pl.BlockSpec(memory_space=pl.ANY),
            pl.BlockSpec(memory_space=pltpu.VMEM),
            pl.BlockSpec(memory_space=pltpu.VMEM),
            pl.BlockSpec(memory_space=pltpu.VMEM),
            pl.BlockSpec(memory_space=pltpu.VMEM),
            pl.BlockSpec(memory_space=pltpu.VMEM),
            pl.BlockSpec(memory_space=pltpu.VMEM),
        ],
        out_specs=pl.BlockSpec((1, T, W, C), lambda b, t: (b, t, 0, 0)),
        out_shape=jax.ShapeDtypeStruct((B, H, W, C), jnp.bfloat16),
        scratch_shapes=[
            pltpu.VMEM((1, W, C), jnp.float32),
            pltpu.VMEM((1, W, C), jnp.float32),
            pltpu.SemaphoreType.DMA((2,)),
        ],
        compiler_params=pltpu.CompilerParams(
            dimension_semantics=("arbitrary", "arbitrary"),
            vmem_limit_bytes=56 * 1024 * 1024),
    )(x, x, stats, hrow, hcol, hcorn, k, Wp)
    return out
